# baseline (speedup 1.0000x reference)
import functools
import jax
import jax.numpy as jnp
from jax import lax
from jax.experimental import pallas as pl
from jax.experimental.pallas import tpu as pltpu
from jax.experimental.pallas import tpu_sc as plsc

F32 = jnp.float32

_NC = 2
_NS = 16
_NW = _NC * _NS
_C = 80


def _ln_exact(x, s, b):
    mu = jnp.mean(x, axis=-1, keepdims=True)
    xc = x - mu
    var = jnp.mean(xc * xc, axis=-1, keepdims=True)
    return xc / jnp.sqrt(var + 1e-5) * s + b


def _row2(a):
    return a.reshape(1, -1)


def _dot(a, b):
    return jnp.dot(a, b, preferred_element_type=F32)


def _mlp_call(terms, b1, W2, b2, ln_s, ln_b, W3=None, b3=None, block=2000,
              rows=None, row0=0):
    first = terms[0]
    R = (first[1][0] if first[0] == "dot" else first[1]).shape[0]
    if rows is None:
        rows = R - row0
    assert rows % block == 0 and row0 % block == 0, (rows, row0, block)
    grid = rows // block
    off = row0 // block
    has_head = W3 is not None
    d_out = (W3.shape[1] if has_head else W2.shape[1])

    arrays = []
    specs = []
    layout = []
    full = lambda a: pl.BlockSpec(a.shape, lambda i: (0, 0))

    def blk(a):
        off_a = 0 if a.shape[0] == rows else off
        return pl.BlockSpec((block, a.shape[1]),
                            lambda i, off_a=off_a: (i + off_a, 0))
    for t in terms:
        if t[0] in ("dot", "dotsum"):
            xi = []
            for x in t[1]:
                xi.append(len(arrays))
                arrays.append(x)
                specs.append(blk(x))
            wi = len(arrays)
            arrays.append(t[2])
            specs.append(full(t[2]))
            layout.append((t[0], xi, wi))
        else:
            ai = len(arrays)
            arrays.append(t[1])
            specs.append(blk(t[1]))
            layout.append(("add", ai))
    tail = [_row2(b1), W2, _row2(b2), _row2(ln_s), _row2(ln_b)]
    if has_head:
        tail += [W3, _row2(b3)]
    tail_base = len(arrays)
    for a in tail:
        arrays.append(a)
        specs.append(full(a))

    def body(*refs):
        h = None
        for t in layout:
            if t[0] == "dot":
                xs = [refs[i][...] for i in t[1]]
                x = xs[0] if len(xs) == 1 else jnp.concatenate(xs, axis=1)
                d = _dot(x, refs[t[2]][...])
            elif t[0] == "dotsum":
                x = refs[t[1][0]][...]
                for i in t[1][1:]:
                    x = x + refs[i][...]
                d = _dot(x, refs[t[2]][...])
            else:
                d = refs[t[1]][...]
            h = d if h is None else h + d
        b1_ref, w2_ref, b2_ref, s_ref, bb_ref = refs[tail_base:tail_base + 5]
        h = jnp.maximum(h + b1_ref[...], 0.0)
        y = jnp.maximum(_dot(h, w2_ref[...]) + b2_ref[...], 0.0)
        y = _ln_exact(y, s_ref[...], bb_ref[...])
        if has_head:
            w3_ref, b3_ref = refs[tail_base + 5:tail_base + 7]
            out_ref = refs[tail_base + 7]
            out_ref[...] = _dot(y, w3_ref[...]) + b3_ref[...]
        else:
            out_ref = refs[tail_base + 5]
            out_ref[...] = y

    return pl.pallas_call(
        body,
        grid=(grid,),
        in_specs=specs,
        out_specs=pl.BlockSpec((block, d_out), lambda i: (i, 0)),
        out_shape=jax.ShapeDtypeStruct((rows, d_out), F32),
    )(*arrays)


def _sc_gather(T, idxr3, idxs3):
    nw, nchunk, c = idxr3.shape
    E = nw * nchunk * c
    mesh = plsc.VectorSubcoreMesh(core_axis_name="c", subcore_axis_name="s")

    nring = 4

    @functools.partial(
        pl.kernel,
        out_type=[jax.ShapeDtypeStruct((E, 128), F32),
                  jax.ShapeDtypeStruct((E, 128), F32)],
        mesh=mesh,
        scratch_types=[
            pltpu.VMEM((nchunk, c), jnp.int32),
            pltpu.VMEM((nchunk, c), jnp.int32),
            pltpu.VMEM((nring, c, 128), F32),
            pltpu.VMEM((nring, c, 128), F32),
        ] + [pltpu.SemaphoreType.DMA] * (4 * nring),
    )
    def k(T_h, ir_h, is_h, gr_h, gs_h, ir_v, is_v, br, bs, *sems):
        gsem_r = sems[0:nring]
        gsem_s = sems[nring:2 * nring]
        wsem_r = sems[2 * nring:3 * nring]
        wsem_s = sems[3 * nring:4 * nring]
        ci = lax.axis_index("c")
        si = lax.axis_index("s")
        wid = si * _NC + ci
        pltpu.sync_copy(ir_h.at[wid], ir_v)
        pltpu.sync_copy(is_h.at[wid], is_v)
        base = wid * (nchunk * c)

        def start_g(j, b):
            pltpu.make_async_copy(T_h.at[ir_v.at[j]], br.at[b], gsem_r[b]).start()
            pltpu.make_async_copy(T_h.at[is_v.at[j]], bs.at[b], gsem_s[b]).start()

        def wait_g(j, b):
            pltpu.make_async_copy(T_h.at[ir_v.at[j]], br.at[b], gsem_r[b]).wait()
            pltpu.make_async_copy(T_h.at[is_v.at[j]], bs.at[b], gsem_s[b]).wait()

        def start_w(j, b):
            dst_r = gr_h.at[pl.ds(base + j * c, c)]
            dst_s = gs_h.at[pl.ds(base + j * c, c)]
            pltpu.make_async_copy(br.at[b], dst_r, wsem_r[b]).start()
            pltpu.make_async_copy(bs.at[b], dst_s, wsem_s[b]).start()

        def wait_w(j, b):
            dst_r = gr_h.at[pl.ds(base + j * c, c)]
            dst_s = gs_h.at[pl.ds(base + j * c, c)]
            pltpu.make_async_copy(br.at[b], dst_r, wsem_r[b]).wait()
            pltpu.make_async_copy(bs.at[b], dst_s, wsem_s[b]).wait()

        start_g(0, 0)
        start_g(1, 1)

        def body(j4, carry):
            for b4 in range(4):
                j = 4 * j4 + b4
                b = b4
                wait_g(j, b)
                start_w(j, b)
                nb = (b + 2) % nring

                @pl.when(j + 2 < nchunk)
                def _():
                    @pl.when(j >= 2)
                    def _():
                        wait_w(j - 2, nb)

                    start_g(j + 2, nb)
            return carry

        lax.fori_loop(0, nchunk // 4, body, 0)
        for jt in range(nchunk - nchunk % 4, nchunk):
            b = jt % nring
            wait_g(jt, b)
            start_w(jt, b)
        for jt in range(nchunk - nring, nchunk):
            wait_w(jt, jt % nring)

    return k(T, idxr3, idxs3)


def _sc_scatter_add(e_new, col3, zeros):
    nw, nchunk, c = col3.shape
    N = zeros.shape[0]
    mesh = plsc.VectorSubcoreMesh(core_axis_name="c", subcore_axis_name="s")

    @functools.partial(
        pl.kernel,
        out_type=jax.ShapeDtypeStruct((2, N, 128), F32),
        mesh=mesh,
        scratch_types=[
            pltpu.VMEM((nchunk, c), jnp.int32),
            pltpu.VMEM((2, c, 128), F32),
            pltpu.VMEM_SHARED((N, 128), F32),
        ] + [pltpu.SemaphoreType.DMA] * 2,
    )
    def k(e_h, col_h, z_h, out_h, col_v, buf, acc, *sems):
        ci = lax.axis_index("c")
        si = lax.axis_index("s")
        wid = si * _NC + ci

        @pl.when(si == 0)
        def _():
            pltpu.sync_copy(z_h, acc)

        pltpu.sync_copy(col_h.at[wid], col_v)
        plsc.subcore_barrier()
        base = wid * (nchunk * c)

        def start_l(j, b):
            pltpu.make_async_copy(e_h.at[pl.ds(base + j * c, c)],
                                  buf.at[b], sems[b]).start()

        def finish(j, b):
            pltpu.make_async_copy(e_h.at[pl.ds(base + j * c, c)],
                                  buf.at[b], sems[b]).wait()
            pltpu.sync_copy(buf.at[b], acc.at[col_v.at[j]], add=True)

        start_l(0, 0)
        start_l(1, 1)

        def body(j2, carry):
            for b in range(2):
                j = 2 * j2 + b
                finish(j, b)

                @pl.when(j + 2 < nchunk)
                def _():
                    start_l(j + 2, b)
            return carry

        lax.fori_loop(0, nchunk // 2, body, 0)
        if nchunk % 2:
            finish(nchunk - 1, 0)
        plsc.subcore_barrier()

        @pl.when(si == 0)
        def _():
            pltpu.sync_copy(acc, out_h.at[ci])

    return k(e_new, col3, zeros)


def _pair_proj(n0, n, Wr, Ws, block=2000):
    N = n0.shape[0]
    grid = N // block

    def body(n0_ref, n_ref, wr_ref, ws_ref, out_ref):
        x = jnp.concatenate([n0_ref[...], n_ref[...]], axis=1)
        out_ref[0] = _dot(x, wr_ref[...])
        out_ref[1] = _dot(x, ws_ref[...])

    full = lambda a: pl.BlockSpec(a.shape, lambda i: (0, 0))
    out = pl.pallas_call(
        body,
        grid=(grid,),
        in_specs=[pl.BlockSpec((block, 128), lambda i: (i, 0))] * 2
        + [full(Wr), full(Ws)],
        out_specs=pl.BlockSpec((2, block, 128), lambda i: (0, i, 0)),
        out_shape=jax.ShapeDtypeStruct((2, N, 128), F32),
    )(n0, n, Wr, Ws)
    return out.reshape(2 * N, 128)


def kernel(edge_attr, node_attr, edge_index, batch, params):
    N = node_attr.shape[0]
    row = edge_index[0]
    col = edge_index[1]

    def mlp_parts(p):
        l1, l2 = p["layers"]
        return l1["W"], l1["b"], l2["W"], l2["b"], p["ln_s"], p["ln_b"]

    W1, b1, W2, b2, s, bb = mlp_parts(params["enc_edge"])
    e0 = _mlp_call([("dot", [edge_attr], W1)], b1, W2, b2, s, bb, block=4000)
    W1, b1, W2, b2, s, bb = mlp_parts(params["enc_node"])
    n0 = _mlp_call([("dot", [node_attr], W1)], b1, W2, b2, s, bb, block=2000)

    n = n0

    E = row.shape[0]
    H = E // 2
    ch = 40
    nch = H // (_NW * ch)
    idx_r = col
    idx_s = row + N
    halves = []
    for h0 in (0, H):
        halves.append((
            lax.dynamic_slice_in_dim(idx_r, h0, H).reshape(_NW, nch, ch),
            lax.dynamic_slice_in_dim(idx_s, h0, H).reshape(_NW, nch, ch),
            lax.dynamic_slice_in_dim(col, h0, H).reshape(_NW, nch, ch),
            h0,
        ))
    zeros = jnp.zeros((N, 128), F32)

    eh = (e0, e0)

    for i in range(5):
        pe = params["proc"][i]["edge"]
        pn = params["proc"][i]["node"]
        We1, be1, We2, be2, se, bbe = mlp_parts(pe)
        Wr, Ws, Wee = We1[0:256], We1[256:512], We1[512:768]

        T = _pair_proj(n0, n, Wr, Ws)
        gs = [_sc_gather(T, hr, hs) for hr, hs, _, _ in halves]
        new_eh = []
        aggps = []
        for hidx, (g_pair, (hr, hs, hc, h0)) in enumerate(zip(gs, halves)):
            g_r, g_s = g_pair
            eh_i = _mlp_call(
                [("add", g_r), ("add", g_s), ("dot", [e0, eh[hidx]], Wee)],
                be1, We2, be2, se, bbe, block=4000, rows=H, row0=h0)
            new_eh.append(eh_i)
            aggps.append(_sc_scatter_add(eh_i, hc, zeros))
        eh = tuple(new_eh)

        Wn1, bn1, Wn2, bn2, sn, bbn = mlp_parts(pn)
        n = _mlp_call([("dot", [n0, n], Wn1[0:256]),
                       ("dotsum", [aggps[0][0], aggps[0][1],
                                   aggps[1][0], aggps[1][1]], Wn1[256:384])],
                      bn1, Wn2, bn2, sn, bbn, block=2000)

    W1, b1, W2, b2, s, bb = mlp_parts(params["dec_edge"])
    W3 = params["out_edge"]["layers"][0]["W"]
    b3 = params["out_edge"]["layers"][0]["b"]
    e_op = jnp.concatenate(
        [_mlp_call([("dot", [ehi], W1)], b1, W2, b2, s, bb, W3=W3, b3=b3,
                   block=4000) for ehi in eh], axis=0)

    W1, b1, W2, b2, s, bb = mlp_parts(params["dec_node"])
    W3 = params["out_node"]["layers"][0]["W"]
    b3 = params["out_node"]["layers"][0]["b"]
    n_op = _mlp_call([("dot", [n], W1)], b1, W2, b2, s, bb, W3=W3, b3=b3,
                     block=2000)

    return (e_op, n_op)


# speedup vs baseline: 1.0532x; 1.0532x over previous
import functools
import jax
import jax.numpy as jnp
from jax import lax
from jax.experimental import pallas as pl
from jax.experimental.pallas import tpu as pltpu
from jax.experimental.pallas import tpu_sc as plsc

F32 = jnp.float32

_NC = 2
_NS = 16
_NW = _NC * _NS
_C = 80


def _ln_exact(x, s, b):
    mu = jnp.mean(x, axis=-1, keepdims=True)
    xc = x - mu
    var = jnp.mean(xc * xc, axis=-1, keepdims=True)
    return xc / jnp.sqrt(var + 1e-5) * s + b


def _row2(a):
    return a.reshape(1, -1)


def _dot(a, b):
    return jnp.dot(a, b, preferred_element_type=F32)


def _mlp_call(terms, b1, W2, b2, ln_s, ln_b, W3=None, b3=None, block=2000):
    first = terms[0]
    R = (first[1][0] if first[0] == "dot" else first[1]).shape[0]
    assert R % block == 0, (R, block)
    grid = R // block
    has_head = W3 is not None
    d_out = (W3.shape[1] if has_head else W2.shape[1])

    arrays = []
    specs = []
    layout = []
    full = lambda a: pl.BlockSpec(a.shape, lambda i: (0, 0))
    blk = lambda a: pl.BlockSpec((block, a.shape[1]), lambda i: (i, 0))
    for t in terms:
        if t[0] in ("dot", "dotsum"):
            xi = []
            for x in t[1]:
                xi.append(len(arrays))
                arrays.append(x)
                specs.append(blk(x))
            wi = len(arrays)
            arrays.append(t[2])
            specs.append(full(t[2]))
            layout.append((t[0], xi, wi))
        else:
            ai = len(arrays)
            arrays.append(t[1])
            specs.append(blk(t[1]))
            layout.append(("add", ai))
    tail = [_row2(b1), W2, _row2(b2), _row2(ln_s), _row2(ln_b)]
    if has_head:
        tail += [W3, _row2(b3)]
    tail_base = len(arrays)
    for a in tail:
        arrays.append(a)
        specs.append(full(a))

    def body(*refs):
        h = None
        for t in layout:
            if t[0] == "dot":
                xs = [refs[i][...] for i in t[1]]
                x = xs[0] if len(xs) == 1 else jnp.concatenate(xs, axis=1)
                d = _dot(x, refs[t[2]][...])
            elif t[0] == "dotsum":
                x = refs[t[1][0]][...]
                for i in t[1][1:]:
                    x = x + refs[i][...]
                d = _dot(x, refs[t[2]][...])
            else:
                d = refs[t[1]][...]
            h = d if h is None else h + d
        b1_ref, w2_ref, b2_ref, s_ref, bb_ref = refs[tail_base:tail_base + 5]
        h = jnp.maximum(h + b1_ref[...], 0.0)
        y = jnp.maximum(_dot(h, w2_ref[...]) + b2_ref[...], 0.0)
        y = _ln_exact(y, s_ref[...], bb_ref[...])
        if has_head:
            w3_ref, b3_ref = refs[tail_base + 5:tail_base + 7]
            out_ref = refs[tail_base + 7]
            out_ref[...] = _dot(y, w3_ref[...]) + b3_ref[...]
        else:
            out_ref = refs[tail_base + 5]
            out_ref[...] = y

    return pl.pallas_call(
        body,
        grid=(grid,),
        in_specs=specs,
        out_specs=pl.BlockSpec((block, d_out), lambda i: (i, 0)),
        out_shape=jax.ShapeDtypeStruct((R, d_out), F32),
    )(*arrays)


def _sc_gather(T, idxr3, idxs3):
    nw, nchunk, c = idxr3.shape
    E = nw * nchunk * c
    mesh = plsc.VectorSubcoreMesh(core_axis_name="c", subcore_axis_name="s")

    nring = 4

    @functools.partial(
        pl.kernel,
        out_type=[jax.ShapeDtypeStruct((E, 128), F32),
                  jax.ShapeDtypeStruct((E, 128), F32)],
        mesh=mesh,
        scratch_types=[
            pltpu.VMEM((nchunk, c), jnp.int32),
            pltpu.VMEM((nchunk, c), jnp.int32),
            pltpu.VMEM((nring, c, 128), F32),
            pltpu.VMEM((nring, c, 128), F32),
        ] + [pltpu.SemaphoreType.DMA] * (4 * nring),
    )
    def k(T_h, ir_h, is_h, gr_h, gs_h, ir_v, is_v, br, bs, *sems):
        gsem_r = sems[0:nring]
        gsem_s = sems[nring:2 * nring]
        wsem_r = sems[2 * nring:3 * nring]
        wsem_s = sems[3 * nring:4 * nring]
        ci = lax.axis_index("c")
        si = lax.axis_index("s")
        wid = si * _NC + ci
        pltpu.sync_copy(ir_h.at[wid], ir_v)
        pltpu.sync_copy(is_h.at[wid], is_v)
        base = wid * (nchunk * c)

        def start_g(j, b):
            pltpu.make_async_copy(T_h.at[ir_v.at[j]], br.at[b], gsem_r[b]).start()
            pltpu.make_async_copy(T_h.at[is_v.at[j]], bs.at[b], gsem_s[b]).start()

        def wait_g(j, b):
            pltpu.make_async_copy(T_h.at[ir_v.at[j]], br.at[b], gsem_r[b]).wait()
            pltpu.make_async_copy(T_h.at[is_v.at[j]], bs.at[b], gsem_s[b]).wait()

        def start_w(j, b):
            dst_r = gr_h.at[pl.ds(base + j * c, c)]
            dst_s = gs_h.at[pl.ds(base + j * c, c)]
            pltpu.make_async_copy(br.at[b], dst_r, wsem_r[b]).start()
            pltpu.make_async_copy(bs.at[b], dst_s, wsem_s[b]).start()

        def wait_w(j, b):
            dst_r = gr_h.at[pl.ds(base + j * c, c)]
            dst_s = gs_h.at[pl.ds(base + j * c, c)]
            pltpu.make_async_copy(br.at[b], dst_r, wsem_r[b]).wait()
            pltpu.make_async_copy(bs.at[b], dst_s, wsem_s[b]).wait()

        start_g(0, 0)
        start_g(1, 1)

        def body(j4, carry):
            for b4 in range(4):
                j = 4 * j4 + b4
                b = b4
                wait_g(j, b)
                start_w(j, b)
                nb = (b + 2) % nring

                @pl.when(j + 2 < nchunk)
                def _():
                    @pl.when(j >= 2)
                    def _():
                        wait_w(j - 2, nb)

                    start_g(j + 2, nb)
            return carry

        lax.fori_loop(0, nchunk // 4, body, 0)
        for jt in range(nchunk - nchunk % 4, nchunk):
            b = jt % nring
            wait_g(jt, b)
            start_w(jt, b)
        for jt in range(nchunk - nring, nchunk):
            wait_w(jt, jt % nring)

    return k(T, idxr3, idxs3)


def _sc_scatter_add(e_new, col3, zeros):
    nw, nchunk, c = col3.shape
    N = zeros.shape[0]
    mesh = plsc.VectorSubcoreMesh(core_axis_name="c", subcore_axis_name="s")

    @functools.partial(
        pl.kernel,
        out_type=jax.ShapeDtypeStruct((2, N, 128), F32),
        mesh=mesh,
        scratch_types=[
            pltpu.VMEM((nchunk, c), jnp.int32),
            pltpu.VMEM((2, c, 128), F32),
            pltpu.VMEM_SHARED((N, 128), F32),
        ] + [pltpu.SemaphoreType.DMA] * 2,
    )
    def k(e_h, col_h, z_h, out_h, col_v, buf, acc, *sems):
        ci = lax.axis_index("c")
        si = lax.axis_index("s")
        wid = si * _NC + ci

        @pl.when(si == 0)
        def _():
            pltpu.sync_copy(z_h, acc)

        pltpu.sync_copy(col_h.at[wid], col_v)
        plsc.subcore_barrier()
        base = wid * (nchunk * c)

        def start_l(j, b):
            pltpu.make_async_copy(e_h.at[pl.ds(base + j * c, c)],
                                  buf.at[b], sems[b]).start()

        def finish(j, b):
            pltpu.make_async_copy(e_h.at[pl.ds(base + j * c, c)],
                                  buf.at[b], sems[b]).wait()
            pltpu.sync_copy(buf.at[b], acc.at[col_v.at[j]], add=True)

        start_l(0, 0)
        start_l(1, 1)

        def body(j2, carry):
            for b in range(2):
                j = 2 * j2 + b
                finish(j, b)

                @pl.when(j + 2 < nchunk)
                def _():
                    start_l(j + 2, b)
            return carry

        lax.fori_loop(0, nchunk // 2, body, 0)
        if nchunk % 2:
            finish(nchunk - 1, 0)
        plsc.subcore_barrier()

        @pl.when(si == 0)
        def _():
            pltpu.sync_copy(acc, out_h.at[ci])

    return k(e_new, col3, zeros)


def _pair_proj(n0, n, Wr, Ws, block=2000):
    N = n0.shape[0]
    grid = N // block

    def body(n0_ref, n_ref, wr_ref, ws_ref, out_ref):
        x = jnp.concatenate([n0_ref[...], n_ref[...]], axis=1)
        out_ref[0] = _dot(x, wr_ref[...])
        out_ref[1] = _dot(x, ws_ref[...])

    full = lambda a: pl.BlockSpec(a.shape, lambda i: (0, 0))
    out = pl.pallas_call(
        body,
        grid=(grid,),
        in_specs=[pl.BlockSpec((block, 128), lambda i: (i, 0))] * 2
        + [full(Wr), full(Ws)],
        out_specs=pl.BlockSpec((2, block, 128), lambda i: (0, i, 0)),
        out_shape=jax.ShapeDtypeStruct((2, N, 128), F32),
    )(n0, n, Wr, Ws)
    return out.reshape(2 * N, 128)


def kernel(edge_attr, node_attr, edge_index, batch, params):
    N = node_attr.shape[0]
    row = edge_index[0]
    col = edge_index[1]

    def mlp_parts(p):
        l1, l2 = p["layers"]
        return l1["W"], l1["b"], l2["W"], l2["b"], p["ln_s"], p["ln_b"]

    W1, b1, W2, b2, s, bb = mlp_parts(params["enc_edge"])
    e0 = _mlp_call([("dot", [edge_attr], W1)], b1, W2, b2, s, bb, block=8000)
    W1, b1, W2, b2, s, bb = mlp_parts(params["enc_node"])
    n0 = _mlp_call([("dot", [node_attr], W1)], b1, W2, b2, s, bb, block=2000)

    e = e0
    n = n0

    E = row.shape[0]
    nchunk = E // (_NW * _C)
    idxr3 = col.reshape(_NW, nchunk, _C)
    idxs3 = (row + N).reshape(_NW, nchunk, _C)
    col3 = col.reshape(_NW, nchunk, _C)
    zeros = jnp.zeros((N, 128), F32)
    ident = jnp.arange(_C, dtype=jnp.int32).reshape(1, _C)

    for i in range(5):
        pe = params["proc"][i]["edge"]
        pn = params["proc"][i]["node"]
        We1, be1, We2, be2, se, bbe = mlp_parts(pe)
        Wr, Ws, Wee = We1[0:256], We1[256:512], We1[512:768]

        T = _pair_proj(n0, n, Wr, Ws)
        g_r, g_s = _sc_gather(T, idxr3, idxs3)

        e = _mlp_call([("add", g_r), ("add", g_s), ("dot", [e0, e], Wee)],
                      be1, We2, be2, se, bbe, block=8000)

        aggp = _sc_scatter_add(e, col3, zeros)

        Wn1, bn1, Wn2, bn2, sn, bbn = mlp_parts(pn)
        n = _mlp_call([("dot", [n0, n], Wn1[0:256]),
                       ("dotsum", [aggp[0], aggp[1]], Wn1[256:384])],
                      bn1, Wn2, bn2, sn, bbn, block=2000)

    W1, b1, W2, b2, s, bb = mlp_parts(params["dec_edge"])
    W3 = params["out_edge"]["layers"][0]["W"]
    b3 = params["out_edge"]["layers"][0]["b"]
    e_op = _mlp_call([("dot", [e], W1)], b1, W2, b2, s, bb, W3=W3, b3=b3,
                     block=8000)

    W1, b1, W2, b2, s, bb = mlp_parts(params["dec_node"])
    W3 = params["out_node"]["layers"][0]["W"]
    b3 = params["out_node"]["layers"][0]["b"]
    n_op = _mlp_call([("dot", [n], W1)], b1, W2, b2, s, bb, W3=W3, b3=b3,
                     block=2000)

    return (e_op, n_op)
